# Initial kernel scaffold; baseline (speedup 1.0000x reference)
import jax
import jax.numpy as jnp
from jax.experimental import pallas as pl


def kernel(x, edge_types, means, temps, mul_weight, bias_weight):
    raise NotImplementedError("write your pallas kernel here")


# speedup vs baseline: 9.9099x; 9.9099x over previous
import functools

import jax
import jax.numpy as jnp
from jax import lax
from jax.experimental import pallas as pl
from jax.experimental.pallas import tpu as pltpu
from jax.experimental.pallas import tpu_sc as plsc

_NUM_CORES = 2
_NUM_SUBCORES = 16
_LANES = 16

_LOG2E = 1.4426950408889634


def _sc_xe(x, edge_types, mul_tbl, bias_tbl):
    e_total = x.shape[0]
    t = mul_tbl.shape[0]
    nw = _NUM_CORES * _NUM_SUBCORES
    chunk = e_total // nw
    assert e_total % nw == 0 and chunk % _LANES == 0 and chunk % 8 == 0

    mesh = plsc.VectorSubcoreMesh(
        core_axis_name="c", subcore_axis_name="s",
        num_cores=_NUM_CORES, num_subcores=_NUM_SUBCORES)

    @functools.partial(
        pl.kernel,
        out_type=jax.ShapeDtypeStruct((e_total,), jnp.float32),
        mesh=mesh,
        scratch_types=[
            pltpu.VMEM((chunk,), jnp.int32),
            pltpu.VMEM((chunk,), jnp.float32),
            pltpu.VMEM((chunk,), jnp.float32),
            pltpu.VMEM((t,), jnp.float32),
            pltpu.VMEM((t,), jnp.float32),
        ],
        compiler_params=pltpu.CompilerParams(needs_layout_passes=False),
    )
    def body(x_hbm, et_hbm, mul_hbm, bias_hbm, xe_hbm,
             idx_v, x_v, xe_v, mul_v, bias_v):
        wid = lax.axis_index("s") * _NUM_CORES + lax.axis_index("c")
        base = wid * chunk
        pltpu.sync_copy(mul_hbm, mul_v)
        pltpu.sync_copy(bias_hbm, bias_v)
        pltpu.sync_copy(et_hbm.at[pl.ds(base, chunk)], idx_v)
        pltpu.sync_copy(x_hbm.at[pl.ds(base, chunk)], x_v)

        def step(i, carry):
            s = pl.ds(i * _LANES, _LANES)
            idx = idx_v[s]
            m = plsc.load_gather(mul_v, [idx])
            b = plsc.load_gather(bias_v, [idx])
            xe_v[s] = m * x_v[s] + b
            return carry

        lax.fori_loop(0, chunk // _LANES, step, 0)
        pltpu.sync_copy(xe_v, xe_hbm.at[pl.ds(base, chunk)])

    return body(x, edge_types, mul_tbl, bias_tbl)


def _tc_rbf_body(means_ref, temps_ref, xe_ref, out_ref):
    m = means_ref[...]
    tmp = jnp.abs(temps_ref[...])
    a = tmp * (-_LOG2E)
    b = (2.0 * _LOG2E) * tmp * m
    c = a * m * m
    xe = xe_ref[...]
    u = xe * xe
    out_ref[...] = jnp.exp2(u * a + (xe * b + c))


def _tc_rbf(xe, means, temps, block_e):
    e_total = xe.shape[0]
    k = means.shape[0]
    assert e_total % block_e == 0
    grid = (e_total // block_e,)
    return pl.pallas_call(
        _tc_rbf_body,
        grid=grid,
        in_specs=[
            pl.BlockSpec((1, k), lambda i: (0, 0)),
            pl.BlockSpec((1, k), lambda i: (0, 0)),
            pl.BlockSpec((block_e, 1), lambda i: (i, 0)),
        ],
        out_specs=pl.BlockSpec((block_e, k), lambda i: (i, 0)),
        out_shape=jax.ShapeDtypeStruct((e_total, k), jnp.float32),
        compiler_params=pltpu.CompilerParams(
            dimension_semantics=("arbitrary",),
        ),
    )(means.reshape(1, k), temps.reshape(1, k), xe.reshape(e_total, 1))


def kernel(x, edge_types, means, temps, mul_weight, bias_weight):
    et = edge_types.astype(jnp.int32)
    mul_tbl = mul_weight.reshape(-1).astype(jnp.float32)
    bias_tbl = bias_weight.reshape(-1).astype(jnp.float32)
    xe = _sc_xe(x.astype(jnp.float32), et, mul_tbl, bias_tbl)
    out = _tc_rbf(xe, means.astype(jnp.float32), temps.astype(jnp.float32),
                  block_e=2000)
    return out.astype(means.dtype)


# speedup vs baseline: 15.9047x; 1.6049x over previous
import functools

import jax
import jax.numpy as jnp
from jax import lax
from jax.experimental import pallas as pl
from jax.experimental.pallas import tpu as pltpu
from jax.experimental.pallas import tpu_sc as plsc

_NUM_CORES = 2
_NUM_SUBCORES = 16
_LANES = 16

_LOG2E = 1.4426950408889634


def _sc_xe(x, edge_types, mul_tbl, bias_tbl):
    e_total = x.shape[0]
    t = mul_tbl.shape[0]
    nw = _NUM_CORES * _NUM_SUBCORES
    chunk = e_total // nw
    assert e_total % nw == 0 and chunk % _LANES == 0 and chunk % 8 == 0

    mesh = plsc.VectorSubcoreMesh(
        core_axis_name="c", subcore_axis_name="s",
        num_cores=_NUM_CORES, num_subcores=_NUM_SUBCORES)

    @functools.partial(
        pl.kernel,
        out_type=jax.ShapeDtypeStruct((e_total,), jnp.float32),
        mesh=mesh,
        scratch_types=[
            pltpu.VMEM((chunk,), jnp.int32),
            pltpu.VMEM((chunk,), jnp.float32),
            pltpu.VMEM((chunk,), jnp.float32),
            pltpu.VMEM((t,), jnp.float32),
            pltpu.VMEM((t,), jnp.float32),
        ],
        compiler_params=pltpu.CompilerParams(needs_layout_passes=False),
    )
    def body(x_hbm, et_hbm, mul_hbm, bias_hbm, xe_hbm,
             idx_v, x_v, xe_v, mul_v, bias_v):
        wid = lax.axis_index("s") * _NUM_CORES + lax.axis_index("c")
        base = wid * chunk
        pltpu.sync_copy(mul_hbm, mul_v)
        pltpu.sync_copy(bias_hbm, bias_v)
        pltpu.sync_copy(et_hbm.at[pl.ds(base, chunk)], idx_v)
        pltpu.sync_copy(x_hbm.at[pl.ds(base, chunk)], x_v)

        def step(i, carry):
            s = pl.ds(i * _LANES, _LANES)
            idx = idx_v[s]
            m = plsc.load_gather(mul_v, [idx])
            b = plsc.load_gather(bias_v, [idx])
            xe_v[s] = m * x_v[s] + b
            return carry

        lax.fori_loop(0, chunk // _LANES, step, 0)
        pltpu.sync_copy(xe_v, xe_hbm.at[pl.ds(base, chunk)])

    return body(x, edge_types, mul_tbl, bias_tbl)


def _tc_rbf_body(means_ref, temps_ref, xe_ref, out_ref):
    m = means_ref[...]
    tmp = jnp.abs(temps_ref[...])
    a = tmp * (-_LOG2E)
    b = (2.0 * _LOG2E) * tmp * m
    c = a * m * m
    xe = xe_ref[0]
    u = xe * xe
    ones = jnp.ones_like(xe)
    lhs = jnp.concatenate([ones, xe, u], axis=0)
    rhs = jnp.concatenate([c, b, a], axis=0)
    poly = lax.dot_general(lhs, rhs, (((0,), (0,)), ((), ())),
                           precision=lax.Precision.HIGHEST,
                           preferred_element_type=jnp.float32)
    out_ref[...] = jnp.exp2(poly)


def _tc_rbf(xe, means, temps, block_e):
    e_total = xe.shape[0]
    k = means.shape[0]
    assert e_total % block_e == 0
    grid = (e_total // block_e,)
    return pl.pallas_call(
        _tc_rbf_body,
        grid=grid,
        in_specs=[
            pl.BlockSpec((1, k), lambda i: (0, 0)),
            pl.BlockSpec((1, k), lambda i: (0, 0)),
            pl.BlockSpec((1, 1, block_e), lambda i: (i, 0, 0)),
        ],
        out_specs=pl.BlockSpec((block_e, k), lambda i: (i, 0)),
        out_shape=jax.ShapeDtypeStruct((e_total, k), jnp.float32),
        compiler_params=pltpu.CompilerParams(
            dimension_semantics=("arbitrary",),
        ),
    )(means.reshape(1, k), temps.reshape(1, k),
      xe.reshape(e_total // block_e, 1, block_e))


def kernel(x, edge_types, means, temps, mul_weight, bias_weight):
    et = edge_types.astype(jnp.int32)
    mul_tbl = mul_weight.reshape(-1).astype(jnp.float32)
    bias_tbl = bias_weight.reshape(-1).astype(jnp.float32)
    xe = _sc_xe(x.astype(jnp.float32), et, mul_tbl, bias_tbl)
    out = _tc_rbf(xe, means.astype(jnp.float32), temps.astype(jnp.float32),
                  block_e=2000)
    return out.astype(means.dtype)


# speedup vs baseline: 19.9496x; 1.2543x over previous
import functools

import jax
import jax.numpy as jnp
from jax import lax
from jax.experimental import pallas as pl
from jax.experimental.pallas import tpu as pltpu
from jax.experimental.pallas import tpu_sc as plsc

_NUM_CORES = 2
_NUM_SUBCORES = 16
_LANES = 16

_LOG2E = 1.4426950408889634


def _sc_xe(x, edge_types, mul_tbl, bias_tbl):
    e_total = x.shape[0]
    t = mul_tbl.shape[0]
    nw = _NUM_CORES * _NUM_SUBCORES
    chunk = e_total // nw
    assert e_total % nw == 0 and chunk % _LANES == 0 and chunk % 8 == 0

    mesh = plsc.VectorSubcoreMesh(
        core_axis_name="c", subcore_axis_name="s",
        num_cores=_NUM_CORES, num_subcores=_NUM_SUBCORES)

    @functools.partial(
        pl.kernel,
        out_type=jax.ShapeDtypeStruct((e_total,), jnp.float32),
        mesh=mesh,
        scratch_types=[
            pltpu.VMEM((chunk,), jnp.int32),
            pltpu.VMEM((chunk,), jnp.float32),
            pltpu.VMEM((chunk,), jnp.float32),
            pltpu.VMEM((t,), jnp.float32),
            pltpu.VMEM((t,), jnp.float32),
        ],
        compiler_params=pltpu.CompilerParams(needs_layout_passes=False),
    )
    def body(x_hbm, et_hbm, mul_hbm, bias_hbm, xe_hbm,
             idx_v, x_v, xe_v, mul_v, bias_v):
        wid = lax.axis_index("s") * _NUM_CORES + lax.axis_index("c")
        base = wid * chunk
        pltpu.sync_copy(mul_hbm, mul_v)
        pltpu.sync_copy(bias_hbm, bias_v)
        pltpu.sync_copy(et_hbm.at[pl.ds(base, chunk)], idx_v)
        pltpu.sync_copy(x_hbm.at[pl.ds(base, chunk)], x_v)

        def step(i, carry):
            s = pl.ds(i * _LANES, _LANES)
            idx = idx_v[s]
            m = plsc.load_gather(mul_v, [idx])
            b = plsc.load_gather(bias_v, [idx])
            xe_v[s] = m * x_v[s] + b
            return carry

        lax.fori_loop(0, chunk // _LANES, step, 0)
        pltpu.sync_copy(xe_v, xe_hbm.at[pl.ds(base, chunk)])

    return body(x, edge_types, mul_tbl, bias_tbl)


def _tc_rbf_body(means_ref, temps_ref, xe_ref, out_ref):
    m = means_ref[...]
    nt = jnp.abs(temps_ref[...]) * (-_LOG2E)
    xe = xe_ref[0]
    hi = xe.astype(jnp.bfloat16).astype(jnp.float32)
    r1 = xe - hi
    mid = r1.astype(jnp.bfloat16).astype(jnp.float32)
    lo = r1 - mid
    lhs = jnp.concatenate([hi, mid, lo], axis=0)
    ones3 = jnp.ones((3, m.shape[1]), jnp.float32)
    xe_b = lax.dot_general(lhs, ones3, (((0,), (0,)), ((), ())),
                           preferred_element_type=jnp.float32)
    d = xe_b - m
    out_ref[...] = jnp.exp2(d * d * nt)


def _tc_rbf(xe, means, temps, block_e):
    e_total = xe.shape[0]
    k = means.shape[0]
    assert e_total % block_e == 0
    grid = (e_total // block_e,)
    return pl.pallas_call(
        _tc_rbf_body,
        grid=grid,
        in_specs=[
            pl.BlockSpec((1, k), lambda i: (0, 0)),
            pl.BlockSpec((1, k), lambda i: (0, 0)),
            pl.BlockSpec((1, 1, block_e), lambda i: (i, 0, 0)),
        ],
        out_specs=pl.BlockSpec((block_e, k), lambda i: (i, 0)),
        out_shape=jax.ShapeDtypeStruct((e_total, k), jnp.float32),
        compiler_params=pltpu.CompilerParams(
            dimension_semantics=("arbitrary",),
        ),
    )(means.reshape(1, k), temps.reshape(1, k),
      xe.reshape(e_total // block_e, 1, block_e))


def kernel(x, edge_types, means, temps, mul_weight, bias_weight):
    et = edge_types.astype(jnp.int32)
    mul_tbl = mul_weight.reshape(-1).astype(jnp.float32)
    bias_tbl = bias_weight.reshape(-1).astype(jnp.float32)
    xe = _sc_xe(x.astype(jnp.float32), et, mul_tbl, bias_tbl)
    out = _tc_rbf(xe, means.astype(jnp.float32), temps.astype(jnp.float32),
                  block_e=2000)
    return out.astype(means.dtype)


# speedup vs baseline: 27.3675x; 1.3718x over previous
import functools

import jax
import jax.numpy as jnp
from jax import lax
from jax.experimental import pallas as pl
from jax.experimental.pallas import tpu as pltpu
from jax.experimental.pallas import tpu_sc as plsc

_NUM_CORES = 2
_NUM_SUBCORES = 16
_LANES = 16

_LOG2E = 1.4426950408889634


def _sc_xe(x, edge_types, mul_tbl, bias_tbl):
    e_total = x.shape[0]
    t = mul_tbl.shape[0]
    nw = _NUM_CORES * _NUM_SUBCORES
    chunk = e_total // nw
    assert e_total % nw == 0 and chunk % _LANES == 0 and chunk % 8 == 0

    mesh = plsc.VectorSubcoreMesh(
        core_axis_name="c", subcore_axis_name="s",
        num_cores=_NUM_CORES, num_subcores=_NUM_SUBCORES)

    @functools.partial(
        pl.kernel,
        out_type=jax.ShapeDtypeStruct((e_total,), jnp.float32),
        mesh=mesh,
        scratch_types=[
            pltpu.VMEM((chunk,), jnp.int32),
            pltpu.VMEM((chunk,), jnp.float32),
            pltpu.VMEM((chunk,), jnp.float32),
            pltpu.VMEM((t,), jnp.float32),
            pltpu.VMEM((t,), jnp.float32),
        ],
        compiler_params=pltpu.CompilerParams(needs_layout_passes=False),
    )
    def body(x_hbm, et_hbm, mul_hbm, bias_hbm, xe_hbm,
             idx_v, x_v, xe_v, mul_v, bias_v):
        wid = lax.axis_index("s") * _NUM_CORES + lax.axis_index("c")
        base = wid * chunk
        pltpu.sync_copy(mul_hbm, mul_v)
        pltpu.sync_copy(bias_hbm, bias_v)
        pltpu.sync_copy(et_hbm.at[pl.ds(base, chunk)], idx_v)
        pltpu.sync_copy(x_hbm.at[pl.ds(base, chunk)], x_v)

        def step(i, carry):
            s = pl.ds(i * _LANES, _LANES)
            idx = idx_v[s]
            m = plsc.load_gather(mul_v, [idx])
            b = plsc.load_gather(bias_v, [idx])
            xe_v[s] = m * x_v[s] + b
            return carry

        lax.fori_loop(0, chunk // _LANES, step, 0)
        pltpu.sync_copy(xe_v, xe_hbm.at[pl.ds(base, chunk)])

    return body(x, edge_types, mul_tbl, bias_tbl)


def _tc_rbf_body(means_ref, temps_ref, xe_ref, out_ref):
    m = means_ref[...]
    nt = jnp.abs(temps_ref[...]) * (-_LOG2E)
    xe = xe_ref[0]
    hi = xe.astype(jnp.bfloat16).astype(jnp.float32)
    r1 = xe - hi
    mid = r1.astype(jnp.bfloat16).astype(jnp.float32)
    lo = r1 - mid
    lhs = jnp.concatenate([hi, mid, lo], axis=0)
    ones3 = jnp.ones((3, m.shape[1]), jnp.float32)
    xe_b = lax.dot_general(lhs, ones3, (((0,), (0,)), ((), ())),
                           preferred_element_type=jnp.float32)
    d = xe_b - m
    out_ref[...] = jnp.exp2(d * d * nt)


def _tc_rbf(xe, means, temps, block_e):
    e_total = xe.shape[0]
    k = means.shape[0]
    assert e_total % block_e == 0
    grid = (e_total // block_e,)
    return pl.pallas_call(
        _tc_rbf_body,
        grid=grid,
        in_specs=[
            pl.BlockSpec((1, k), lambda i: (0, 0)),
            pl.BlockSpec((1, k), lambda i: (0, 0)),
            pl.BlockSpec((1, 1, block_e), lambda i: (i, 0, 0)),
        ],
        out_specs=pl.BlockSpec((block_e, k), lambda i: (i, 0)),
        out_shape=jax.ShapeDtypeStruct((e_total, k), jnp.float32),
        compiler_params=pltpu.CompilerParams(
            dimension_semantics=("arbitrary",),
        ),
    )(means.reshape(1, k), temps.reshape(1, k),
      xe.reshape(e_total // block_e, 1, block_e))


def kernel(x, edge_types, means, temps, mul_weight, bias_weight):
    et = edge_types.astype(jnp.int32)
    mul_tbl = mul_weight.reshape(-1).astype(jnp.float32)
    bias_tbl = bias_weight.reshape(-1).astype(jnp.float32)
    xe = _sc_xe(x.astype(jnp.float32), et, mul_tbl, bias_tbl)
    out = _tc_rbf(xe, means.astype(jnp.float32), temps.astype(jnp.float32),
                  block_e=4000)
    return out.astype(means.dtype)


# speedup vs baseline: 33.0780x; 1.2087x over previous
import functools

import jax
import jax.numpy as jnp
from jax import lax
from jax.experimental import pallas as pl
from jax.experimental.pallas import tpu as pltpu
from jax.experimental.pallas import tpu_sc as plsc

_NUM_CORES = 2
_NUM_SUBCORES = 16
_LANES = 16

_LOG2E = 1.4426950408889634


def _sc_xe(x, edge_types, mul_tbl, bias_tbl):
    e_total = x.shape[0]
    t = mul_tbl.shape[0]
    nw = _NUM_CORES * _NUM_SUBCORES
    chunk = e_total // nw
    assert e_total % nw == 0 and chunk % _LANES == 0 and chunk % 8 == 0

    mesh = plsc.VectorSubcoreMesh(
        core_axis_name="c", subcore_axis_name="s",
        num_cores=_NUM_CORES, num_subcores=_NUM_SUBCORES)

    @functools.partial(
        pl.kernel,
        out_type=jax.ShapeDtypeStruct((e_total,), jnp.float32),
        mesh=mesh,
        scratch_types=[
            pltpu.VMEM((chunk,), jnp.int32),
            pltpu.VMEM((chunk,), jnp.float32),
            pltpu.VMEM((chunk,), jnp.float32),
            pltpu.VMEM((t,), jnp.float32),
            pltpu.VMEM((t,), jnp.float32),
        ],
        compiler_params=pltpu.CompilerParams(needs_layout_passes=False),
    )
    def body(x_hbm, et_hbm, mul_hbm, bias_hbm, xe_hbm,
             idx_v, x_v, xe_v, mul_v, bias_v):
        wid = lax.axis_index("s") * _NUM_CORES + lax.axis_index("c")
        base = wid * chunk
        pltpu.sync_copy(mul_hbm, mul_v)
        pltpu.sync_copy(bias_hbm, bias_v)
        pltpu.sync_copy(et_hbm.at[pl.ds(base, chunk)], idx_v)
        pltpu.sync_copy(x_hbm.at[pl.ds(base, chunk)], x_v)

        def step(i, carry):
            s = pl.ds(i * _LANES, _LANES)
            idx = idx_v[s]
            m = plsc.load_gather(mul_v, [idx])
            b = plsc.load_gather(bias_v, [idx])
            xe_v[s] = m * x_v[s] + b
            return carry

        lax.fori_loop(0, chunk // _LANES, step, 0)
        pltpu.sync_copy(xe_v, xe_hbm.at[pl.ds(base, chunk)])

    return body(x, edge_types, mul_tbl, bias_tbl)


def _tc_rbf_body(means_ref, temps_ref, xe_ref, out_ref):
    m = means_ref[...]
    nt = jnp.abs(temps_ref[...]) * (-_LOG2E)
    xe = xe_ref[0]
    hi = xe.astype(jnp.bfloat16).astype(jnp.float32)
    r1 = xe - hi
    mid = r1.astype(jnp.bfloat16).astype(jnp.float32)
    lo = r1 - mid
    lhs = jnp.concatenate([hi, mid, lo], axis=0)
    ones3 = jnp.ones((3, m.shape[1]), jnp.float32)
    xe_b = lax.dot_general(lhs, ones3, (((0,), (0,)), ((), ())),
                           preferred_element_type=jnp.float32)
    d = xe_b - m
    out_ref[...] = jnp.exp2(d * d * nt)


def _tc_rbf(xe, means, temps, block_e):
    e_total = xe.shape[0]
    k = means.shape[0]
    assert e_total % block_e == 0
    grid = (e_total // block_e,)
    return pl.pallas_call(
        _tc_rbf_body,
        grid=grid,
        in_specs=[
            pl.BlockSpec((1, k), lambda i: (0, 0)),
            pl.BlockSpec((1, k), lambda i: (0, 0)),
            pl.BlockSpec((1, 1, block_e), lambda i: (i, 0, 0)),
        ],
        out_specs=pl.BlockSpec((block_e, k), lambda i: (i, 0)),
        out_shape=jax.ShapeDtypeStruct((e_total, k), jnp.float32),
        compiler_params=pltpu.CompilerParams(
            dimension_semantics=("arbitrary",),
        ),
    )(means.reshape(1, k), temps.reshape(1, k),
      xe.reshape(e_total // block_e, 1, block_e))


def kernel(x, edge_types, means, temps, mul_weight, bias_weight):
    et = edge_types.astype(jnp.int32)
    mul_tbl = mul_weight.reshape(-1).astype(jnp.float32)
    bias_tbl = bias_weight.reshape(-1).astype(jnp.float32)
    xe = _sc_xe(x.astype(jnp.float32), et, mul_tbl, bias_tbl)
    out = _tc_rbf(xe, means.astype(jnp.float32), temps.astype(jnp.float32),
                  block_e=8000)
    return out.astype(means.dtype)


# speedup vs baseline: 36.1404x; 1.0926x over previous
import functools

import jax
import jax.numpy as jnp
from jax import lax
from jax.experimental import pallas as pl
from jax.experimental.pallas import tpu as pltpu
from jax.experimental.pallas import tpu_sc as plsc

_NUM_CORES = 2
_NUM_SUBCORES = 16
_LANES = 16

_LOG2E = 1.4426950408889634


def _sc_xe(x, edge_types, mul_tbl, bias_tbl):
    e_total = x.shape[0]
    t = mul_tbl.shape[0]
    nw = _NUM_CORES * _NUM_SUBCORES
    chunk = e_total // nw
    assert e_total % nw == 0 and chunk % _LANES == 0 and chunk % 8 == 0

    mesh = plsc.VectorSubcoreMesh(
        core_axis_name="c", subcore_axis_name="s",
        num_cores=_NUM_CORES, num_subcores=_NUM_SUBCORES)

    @functools.partial(
        pl.kernel,
        out_type=jax.ShapeDtypeStruct((e_total,), jnp.float32),
        mesh=mesh,
        scratch_types=[
            pltpu.VMEM((chunk,), jnp.int32),
            pltpu.VMEM((chunk,), jnp.float32),
            pltpu.VMEM((chunk,), jnp.float32),
            pltpu.VMEM((t,), jnp.float32),
            pltpu.VMEM((t,), jnp.float32),
        ],
        compiler_params=pltpu.CompilerParams(needs_layout_passes=False),
    )
    def body(x_hbm, et_hbm, mul_hbm, bias_hbm, xe_hbm,
             idx_v, x_v, xe_v, mul_v, bias_v):
        wid = lax.axis_index("s") * _NUM_CORES + lax.axis_index("c")
        base = wid * chunk
        pltpu.sync_copy(mul_hbm, mul_v)
        pltpu.sync_copy(bias_hbm, bias_v)
        pltpu.sync_copy(et_hbm.at[pl.ds(base, chunk)], idx_v)
        pltpu.sync_copy(x_hbm.at[pl.ds(base, chunk)], x_v)

        def step(i, carry):
            s = pl.ds(i * _LANES, _LANES)
            idx = idx_v[s]
            m = plsc.load_gather(mul_v, [idx])
            b = plsc.load_gather(bias_v, [idx])
            xe_v[s] = m * x_v[s] + b
            return carry

        lax.fori_loop(0, chunk // _LANES, step, 0)
        pltpu.sync_copy(xe_v, xe_hbm.at[pl.ds(base, chunk)])

    return body(x, edge_types, mul_tbl, bias_tbl)


def _tc_rbf_body(means_ref, temps_ref, xe_ref, out_ref):
    m = means_ref[...]
    nt = jnp.abs(temps_ref[...]) * (-_LOG2E)
    xe = xe_ref[0]
    hi = xe.astype(jnp.bfloat16).astype(jnp.float32)
    r1 = xe - hi
    mid = r1.astype(jnp.bfloat16).astype(jnp.float32)
    lo = r1 - mid
    lhs = jnp.concatenate([hi, mid, lo], axis=0)
    ones3 = jnp.ones((3, m.shape[1]), jnp.float32)
    xe_b = lax.dot_general(lhs, ones3, (((0,), (0,)), ((), ())),
                           preferred_element_type=jnp.float32)
    d = xe_b - m
    out_ref[...] = jnp.exp2(d * d * nt)


def _tc_rbf(xe, means, temps, block_e):
    e_total = xe.shape[0]
    k = means.shape[0]
    assert e_total % block_e == 0
    grid = (e_total // block_e,)
    return pl.pallas_call(
        _tc_rbf_body,
        grid=grid,
        in_specs=[
            pl.BlockSpec((1, k), lambda i: (0, 0)),
            pl.BlockSpec((1, k), lambda i: (0, 0)),
            pl.BlockSpec((1, 1, block_e), lambda i: (i, 0, 0)),
        ],
        out_specs=pl.BlockSpec((block_e, k), lambda i: (i, 0)),
        out_shape=jax.ShapeDtypeStruct((e_total, k), jnp.float32),
        compiler_params=pltpu.CompilerParams(
            dimension_semantics=("arbitrary",),
        ),
    )(means.reshape(1, k), temps.reshape(1, k),
      xe.reshape(e_total // block_e, 1, block_e))


def kernel(x, edge_types, means, temps, mul_weight, bias_weight):
    et = edge_types.astype(jnp.int32)
    mul_tbl = mul_weight.reshape(-1).astype(jnp.float32)
    bias_tbl = bias_weight.reshape(-1).astype(jnp.float32)
    xe = _sc_xe(x.astype(jnp.float32), et, mul_tbl, bias_tbl)
    out = _tc_rbf(xe, means.astype(jnp.float32), temps.astype(jnp.float32),
                  block_e=16000)
    return out.astype(means.dtype)


# speedup vs baseline: 36.6151x; 1.0131x over previous
import functools

import jax
import jax.numpy as jnp
from jax import lax
from jax.experimental import pallas as pl
from jax.experimental.pallas import tpu as pltpu
from jax.experimental.pallas import tpu_sc as plsc

_NUM_CORES = 2
_NUM_SUBCORES = 16
_LANES = 16

_LOG2E = 1.4426950408889634


def _sc_xe(x, edge_types, mul_tbl, bias_tbl):
    e_total = x.shape[0]
    t = mul_tbl.shape[0]
    nw = _NUM_CORES * _NUM_SUBCORES
    chunk = e_total // nw
    assert e_total % nw == 0 and chunk % _LANES == 0 and chunk % 8 == 0

    mesh = plsc.VectorSubcoreMesh(
        core_axis_name="c", subcore_axis_name="s",
        num_cores=_NUM_CORES, num_subcores=_NUM_SUBCORES)

    @functools.partial(
        pl.kernel,
        out_type=jax.ShapeDtypeStruct((e_total,), jnp.float32),
        mesh=mesh,
        scratch_types=[
            pltpu.VMEM((chunk,), jnp.int32),
            pltpu.VMEM((chunk,), jnp.float32),
            pltpu.VMEM((chunk,), jnp.float32),
            pltpu.VMEM((t,), jnp.float32),
            pltpu.VMEM((t,), jnp.float32),
        ],
        compiler_params=pltpu.CompilerParams(needs_layout_passes=False),
    )
    def body(x_hbm, et_hbm, mul_hbm, bias_hbm, xe_hbm,
             idx_v, x_v, xe_v, mul_v, bias_v):
        wid = lax.axis_index("s") * _NUM_CORES + lax.axis_index("c")
        base = wid * chunk
        pltpu.sync_copy(mul_hbm, mul_v)
        pltpu.sync_copy(bias_hbm, bias_v)
        pltpu.sync_copy(et_hbm.at[pl.ds(base, chunk)], idx_v)
        pltpu.sync_copy(x_hbm.at[pl.ds(base, chunk)], x_v)

        def step(i, carry):
            s = pl.ds(i * _LANES, _LANES)
            idx = idx_v[s]
            m = plsc.load_gather(mul_v, [idx])
            b = plsc.load_gather(bias_v, [idx])
            xe_v[s] = m * x_v[s] + b
            return carry

        lax.fori_loop(0, chunk // _LANES, step, 0)
        pltpu.sync_copy(xe_v, xe_hbm.at[pl.ds(base, chunk)])

    return body(x, edge_types, mul_tbl, bias_tbl)


def _tc_rbf_body(means_ref, temps_ref, xe_ref, out_ref):
    m = means_ref[...]
    nt = jnp.abs(temps_ref[...]) * (-_LOG2E)
    xe = xe_ref[0]
    hi = xe.astype(jnp.bfloat16).astype(jnp.float32)
    r1 = xe - hi
    mid = r1.astype(jnp.bfloat16).astype(jnp.float32)
    lo = r1 - mid
    lhs = jnp.concatenate([hi, mid, lo], axis=0)
    ones3 = jnp.ones((3, m.shape[1]), jnp.float32)
    xe_b = lax.dot_general(lhs, ones3, (((0,), (0,)), ((), ())),
                           preferred_element_type=jnp.float32)
    d = xe_b - m
    out_ref[...] = jnp.exp2(d * d * nt)


def _tc_rbf(xe, means, temps, block_e):
    e_total = xe.shape[0]
    k = means.shape[0]
    assert e_total % block_e == 0
    grid = (e_total // block_e,)
    return pl.pallas_call(
        _tc_rbf_body,
        grid=grid,
        in_specs=[
            pl.BlockSpec((1, k), lambda i: (0, 0)),
            pl.BlockSpec((1, k), lambda i: (0, 0)),
            pl.BlockSpec((1, 1, block_e), lambda i: (i, 0, 0)),
        ],
        out_specs=pl.BlockSpec((block_e, k), lambda i: (i, 0)),
        out_shape=jax.ShapeDtypeStruct((e_total, k), jnp.float32),
        compiler_params=pltpu.CompilerParams(
            dimension_semantics=("arbitrary",),
        ),
    )(means.reshape(1, k), temps.reshape(1, k),
      xe.reshape(e_total // block_e, 1, block_e))


def kernel(x, edge_types, means, temps, mul_weight, bias_weight):
    et = edge_types.astype(jnp.int32)
    mul_tbl = mul_weight.reshape(-1).astype(jnp.float32)
    bias_tbl = bias_weight.reshape(-1).astype(jnp.float32)
    xe = _sc_xe(x.astype(jnp.float32), et, mul_tbl, bias_tbl)
    out = _tc_rbf(xe, means.astype(jnp.float32), temps.astype(jnp.float32),
                  block_e=32000)
    return out.astype(means.dtype)


# speedup vs baseline: 38.7794x; 1.0591x over previous
import functools

import jax
import jax.numpy as jnp
from jax import lax
from jax.experimental import pallas as pl
from jax.experimental.pallas import tpu as pltpu
from jax.experimental.pallas import tpu_sc as plsc

_NUM_CORES = 2
_NUM_SUBCORES = 16
_LANES = 16

_LOG2E = 1.4426950408889634


def _sc_xe(x, edge_types, mul_tbl, bias_tbl):
    e_total = x.shape[0]
    t = mul_tbl.shape[0]
    nw = _NUM_CORES * _NUM_SUBCORES
    chunk = e_total // nw
    assert e_total % nw == 0 and chunk % _LANES == 0 and chunk % 8 == 0

    mesh = plsc.VectorSubcoreMesh(
        core_axis_name="c", subcore_axis_name="s",
        num_cores=_NUM_CORES, num_subcores=_NUM_SUBCORES)

    @functools.partial(
        pl.kernel,
        out_type=jax.ShapeDtypeStruct((e_total,), jnp.float32),
        mesh=mesh,
        scratch_types=[
            pltpu.VMEM((chunk,), jnp.int32),
            pltpu.VMEM((chunk,), jnp.float32),
            pltpu.VMEM((chunk,), jnp.float32),
            pltpu.VMEM((t,), jnp.float32),
            pltpu.VMEM((t,), jnp.float32),
            pltpu.SemaphoreType.DMA,
            pltpu.SemaphoreType.DMA,
            pltpu.SemaphoreType.DMA,
            pltpu.SemaphoreType.DMA,
        ],
        compiler_params=pltpu.CompilerParams(needs_layout_passes=False),
    )
    def body(x_hbm, et_hbm, mul_hbm, bias_hbm, xe_hbm,
             idx_v, x_v, xe_v, mul_v, bias_v, sem0, sem1, sem2, sem3):
        wid = lax.axis_index("s") * _NUM_CORES + lax.axis_index("c")
        base = wid * chunk
        c0 = pltpu.async_copy(mul_hbm, mul_v, sem0)
        c1 = pltpu.async_copy(bias_hbm, bias_v, sem1)
        c2 = pltpu.async_copy(et_hbm.at[pl.ds(base, chunk)], idx_v, sem2)
        c3 = pltpu.async_copy(x_hbm.at[pl.ds(base, chunk)], x_v, sem3)
        c0.wait()
        c1.wait()
        c2.wait()
        c3.wait()

        @plsc.parallel_loop(0, chunk // _LANES, unroll=8)
        def step(i):
            s = pl.ds(i * _LANES, _LANES)
            idx = idx_v[s]
            m = plsc.load_gather(mul_v, [idx])
            b = plsc.load_gather(bias_v, [idx])
            xe_v[s] = m * x_v[s] + b

        pltpu.sync_copy(xe_v, xe_hbm.at[pl.ds(base, chunk)])

    return body(x, edge_types, mul_tbl, bias_tbl)


def _tc_rbf_body(means_ref, temps_ref, xe_ref, out_ref):
    m = means_ref[...]
    nt = jnp.abs(temps_ref[...]) * (-_LOG2E)
    xe = xe_ref[0]
    hi = xe.astype(jnp.bfloat16).astype(jnp.float32)
    r1 = xe - hi
    mid = r1.astype(jnp.bfloat16).astype(jnp.float32)
    lo = r1 - mid
    lhs = jnp.concatenate([hi, mid, lo], axis=0)
    ones3 = jnp.ones((3, m.shape[1]), jnp.float32)
    xe_b = lax.dot_general(lhs, ones3, (((0,), (0,)), ((), ())),
                           preferred_element_type=jnp.float32)
    d = xe_b - m
    out_ref[...] = jnp.exp2(d * d * nt)


def _tc_rbf(xe, means, temps, block_e):
    e_total = xe.shape[0]
    k = means.shape[0]
    assert e_total % block_e == 0
    grid = (e_total // block_e,)
    return pl.pallas_call(
        _tc_rbf_body,
        grid=grid,
        in_specs=[
            pl.BlockSpec((1, k), lambda i: (0, 0)),
            pl.BlockSpec((1, k), lambda i: (0, 0)),
            pl.BlockSpec((1, 1, block_e), lambda i: (i, 0, 0)),
        ],
        out_specs=pl.BlockSpec((block_e, k), lambda i: (i, 0)),
        out_shape=jax.ShapeDtypeStruct((e_total, k), jnp.float32),
        compiler_params=pltpu.CompilerParams(
            dimension_semantics=("arbitrary",),
        ),
    )(means.reshape(1, k), temps.reshape(1, k),
      xe.reshape(e_total // block_e, 1, block_e))


def kernel(x, edge_types, means, temps, mul_weight, bias_weight):
    et = edge_types.astype(jnp.int32)
    mul_tbl = mul_weight.reshape(-1).astype(jnp.float32)
    bias_tbl = bias_weight.reshape(-1).astype(jnp.float32)
    xe = _sc_xe(x.astype(jnp.float32), et, mul_tbl, bias_tbl)
    out = _tc_rbf(xe, means.astype(jnp.float32), temps.astype(jnp.float32),
                  block_e=32000)
    return out.astype(means.dtype)
